# baseline (speedup 1.0000x reference)
import functools

import jax
import jax.numpy as jnp
from jax import lax
from jax.experimental import pallas as pl
from jax.experimental.pallas import tpu as pltpu
from jax.experimental.pallas import tpu_sc as plsc

NC = 2
NS = 16
NW = NC * NS
NBUF = 8
L = 6


def _sc_gather(nb, h, d, dtype):
    b_w = nb // NW
    hp = (h + 7) // 8 * 8
    dp = 128
    assert b_w * NW == nb and b_w % NBUF == 0 and b_w > NBUF

    mesh = plsc.VectorSubcoreMesh(
        core_axis_name="c", subcore_axis_name="s",
        num_cores=NC, num_subcores=NS)

    @functools.partial(
        pl.kernel,
        out_type=jax.ShapeDtypeStruct((nb, hp, dp), dtype),
        mesh=mesh,
        scratch_types=[
            pltpu.VMEM((b_w, h), jnp.int32),
            pltpu.VMEM((NBUF, h, d), dtype),
            [pltpu.SemaphoreType.DMA] * NBUF,
            [pltpu.SemaphoreType.DMA] * NBUF,
        ],
        compiler_params=pltpu.CompilerParams(use_tc_tiling_on_sc=False),
    )
    def run(tab_hbm, idx_hbm, out_hbm, idx_v, bufs, gsem, ssem):
        wid = lax.axis_index("s") * NC + lax.axis_index("c")
        b0 = wid * b_w
        pltpu.sync_copy(idx_hbm.at[pl.ds(b0, b_w)], idx_v)

        def fire_gather(j, bf):
            pltpu.async_copy(
                tab_hbm.at[idx_v.at[j]], bufs.at[bf], gsem[bf])

        def wait_gather(j, bf):
            pltpu.make_async_copy(
                tab_hbm.at[idx_v.at[j]], bufs.at[bf], gsem[bf]).wait()

        def fire_out(j, bf):
            pltpu.async_copy(
                bufs.at[bf],
                out_hbm.at[b0 + j, pl.ds(0, h), pl.ds(0, d)],
                ssem[bf])

        def wait_out(bf):
            pltpu.make_async_copy(
                bufs.at[bf],
                out_hbm.at[0, pl.ds(0, h), pl.ds(0, d)],
                ssem[bf]).wait()

        for j in range(L):
            fire_gather(j, j % NBUF)

        for j in range(NBUF - L):
            wait_gather(j, j)
            fire_out(j, j)
            fire_gather(j + L, (j + L) % NBUF)

        g_lo = NBUF - L
        g_hi = b_w - L
        n_steady = ((g_hi - g_lo) // NBUF) * NBUF
        @pl.loop(0, n_steady // NBUF)
        def _step(t):
            for u in range(NBUF):
                j = g_lo + t * NBUF + u
                bf = (g_lo + u) % NBUF
                bn = (bf + L) % NBUF
                wait_gather(j, bf)
                fire_out(j, bf)
                wait_out(bn)
                fire_gather(j + L, bn)

        for j in range(g_lo + n_steady, b_w):
            bf = j % NBUF
            wait_gather(j, bf)
            fire_out(j, bf)
            if j + L < b_w:
                bn = (j + L) % NBUF
                wait_out(bn)
                fire_gather(j + L, bn)

        for bf in range(min(NBUF, b_w)):
            wait_out(bf)

    return run


def kernel(x, table):
    nb, h = x.shape
    n, d = table.shape
    xi = x.astype(jnp.int32)
    q = _sc_gather(nb, h, d, table.dtype)(table, xi)
    return q[:, :h, :d]


# speedup vs baseline: 1.0013x; 1.0013x over previous
import functools

import jax
import jax.numpy as jnp
from jax import lax
from jax.experimental import pallas as pl
from jax.experimental.pallas import tpu as pltpu
from jax.experimental.pallas import tpu_sc as plsc

NC = 2
NS = 16
NW = NC * NS
NBUF = 8
L = 6


def _sc_gather(nb, h, d, dtype):
    b_w = nb // NW
    hp = (h + 7) // 8 * 8
    dp = 128
    assert b_w * NW == nb and (b_w // 2) % NBUF == 0 and b_w // 2 > NBUF

    mesh = plsc.VectorSubcoreMesh(
        core_axis_name="c", subcore_axis_name="s",
        num_cores=NC, num_subcores=NS)

    @functools.partial(
        pl.kernel,
        out_type=jax.ShapeDtypeStruct((nb, hp, dp), dtype),
        mesh=mesh,
        scratch_types=[
            pltpu.VMEM((b_w // 2, 2 * h), jnp.int32),
            pltpu.VMEM((NBUF, 2 * h, d), dtype),
            [pltpu.SemaphoreType.DMA] * NBUF,
            [pltpu.SemaphoreType.DMA] * NBUF,
        ],
        compiler_params=pltpu.CompilerParams(use_tc_tiling_on_sc=False),
    )
    def run(tab_hbm, idx_hbm, out_hbm, idx_v, bufs, gsem, ssem):
        wid = lax.axis_index("s") * NC + lax.axis_index("c")
        b0 = wid * b_w
        pltpu.sync_copy(idx_hbm.at[pl.ds(wid * (b_w // 2), b_w // 2)], idx_v)

        def fire_gather(j, bf):
            pltpu.async_copy(
                tab_hbm.at[idx_v.at[j]], bufs.at[bf], gsem[bf])

        def wait_gather(j, bf):
            pltpu.make_async_copy(
                tab_hbm.at[idx_v.at[j]], bufs.at[bf], gsem[bf]).wait()

        def fire_out(j, bf):
            for e in range(2):
                pltpu.async_copy(
                    bufs.at[bf, pl.ds(e * h, h)],
                    out_hbm.at[b0 + 2 * j + e, pl.ds(0, h), pl.ds(0, d)],
                    ssem[bf])

        def wait_out(bf):
            for e in range(2):
                pltpu.make_async_copy(
                    bufs.at[bf, pl.ds(e * h, h)],
                    out_hbm.at[0, pl.ds(0, h), pl.ds(0, d)],
                    ssem[bf]).wait()

        for j in range(L):
            fire_gather(j, j % NBUF)

        for j in range(NBUF - L):
            wait_gather(j, j)
            fire_out(j, j)
            fire_gather(j + L, (j + L) % NBUF)

        g_lo = NBUF - L
        g_hi = b_w // 2 - L
        n_steady = ((g_hi - g_lo) // NBUF) * NBUF
        @pl.loop(0, n_steady // NBUF)
        def _step(t):
            for u in range(NBUF):
                j = g_lo + t * NBUF + u
                bf = (g_lo + u) % NBUF
                bn = (bf + L) % NBUF
                wait_gather(j, bf)
                fire_out(j, bf)
                wait_out(bn)
                fire_gather(j + L, bn)

        for j in range(g_lo + n_steady, b_w // 2):
            bf = j % NBUF
            wait_gather(j, bf)
            fire_out(j, bf)
            if j + L < b_w // 2:
                bn = (j + L) % NBUF
                wait_out(bn)
                fire_gather(j + L, bn)

        for bf in range(min(NBUF, b_w // 2)):
            wait_out(bf)

    return run


def kernel(x, table):
    nb, h = x.shape
    n, d = table.shape
    xi = x.reshape(nb // 2, 2 * h).astype(jnp.int32)
    q = _sc_gather(nb, h, d, table.dtype)(table, xi)
    return q[:, :h, :d]
